# baseline (speedup 1.0000x reference)
import jax
import jax.numpy as jnp
from jax import lax
from jax.experimental import pallas as pl
from jax.experimental.pallas import tpu as pltpu
from jax.experimental.pallas import tpu_sc as plsc

N_NODES = 100000
N_EDGES = 3200000
NG = 10

NUM_CORES = 2
NUM_SUBCORES = 16
N_WORKERS = NUM_CORES * NUM_SUBCORES
PER_WORKER = N_EDGES // N_WORKERS
CHUNK = 2000
GROUPS = CHUNK // 16
N_CHUNKS = PER_WORKER // CHUNK

_MAGIC = 0x5F3759DF


def _edge_body(ei_hbm, pos_hbm, prm_hbm, out_hbm,
               srcv, dstv, idxi, idxj, pif, pjf, outf, prmv, sem_a, sem_b):
    c = lax.axis_index("c")
    s = lax.axis_index("s")
    wid = s * NUM_CORES + c

    pltpu.sync_copy(prm_hbm, prmv)
    shifts = [prmv[k] for k in range(NG)]
    coefs = [prmv[NG + k] for k in range(NG)]
    iota = lax.iota(jnp.int32, 16)

    def chunk_body(ci, carry):
        base = wid * PER_WORKER + ci * CHUNK
        pltpu.sync_copy(ei_hbm.at[pl.ds(base, CHUNK)], srcv)
        pltpu.sync_copy(ei_hbm.at[pl.ds(N_EDGES + base, CHUNK)], dstv)

        def expand(g, carry2):
            sl = pl.ds(g * 16, 16)
            s3 = srcv[sl] * 3
            d3 = dstv[sl] * 3
            idxi[sl] = s3
            idxi[pl.ds(CHUNK + g * 16, 16)] = s3 + 1
            idxi[pl.ds(2 * CHUNK + g * 16, 16)] = s3 + 2
            idxj[sl] = d3
            idxj[pl.ds(CHUNK + g * 16, 16)] = d3 + 1
            idxj[pl.ds(2 * CHUNK + g * 16, 16)] = d3 + 2
            return carry2

        lax.fori_loop(0, GROUPS, expand, 0)
        cp_i = pltpu.async_copy(pos_hbm.at[idxi], pif, sem_a)
        cp_j = pltpu.async_copy(pos_hbm.at[idxj], pjf, sem_b)
        cp_i.wait()
        cp_j.wait()

        def grp(g, carry2):
            sl = pl.ds(g * 16, 16)
            sly = pl.ds(CHUNK + g * 16, 16)
            slz = pl.ds(2 * CHUNK + g * 16, 16)
            dx = pif[sl] - pjf[sl]
            dy = pif[sly] - pjf[sly]
            dz = pif[slz] - pjf[slz]
            d2 = jnp.maximum(dx * dx + dy * dy + dz * dz,
                             jnp.float32(1e-30))
            bits = lax.bitcast_convert_type(d2, jnp.int32)
            r = lax.bitcast_convert_type(
                jnp.int32(_MAGIC) - lax.shift_right_logical(bits, 1),
                jnp.float32)
            half = jnp.float32(0.5) * d2
            for _ in range(3):
                r = r * (jnp.float32(1.5) - half * r * r)
            length = d2 * r
            orow = (g * 16 + iota) * NG
            for k in range(NG):
                t = length - shifts[k]
                o = jnp.exp(t * t * coefs[k])
                plsc.store_scatter(outf, [orow + k], o)
            return carry2

        lax.fori_loop(0, GROUPS, grp, 0)
        pltpu.sync_copy(outf, out_hbm.at[pl.ds(base * NG, CHUNK * NG)])
        return carry

    lax.fori_loop(0, N_CHUNKS, chunk_body, 0)


@jax.jit
def _gbf_sc(ei, posf, prm):
    mesh = plsc.VectorSubcoreMesh(core_axis_name="c", subcore_axis_name="s")
    fn = pl.kernel(
        _edge_body,
        out_type=jax.ShapeDtypeStruct((N_EDGES * NG,), jnp.float32),
        mesh=mesh,
        compiler_params=pltpu.CompilerParams(needs_layout_passes=False),
        scratch_types=[
            pltpu.VMEM((CHUNK,), jnp.int32),
            pltpu.VMEM((CHUNK,), jnp.int32),
            pltpu.VMEM((3 * CHUNK,), jnp.int32),
            pltpu.VMEM((3 * CHUNK,), jnp.int32),
            pltpu.VMEM((3 * CHUNK,), jnp.float32),
            pltpu.VMEM((3 * CHUNK,), jnp.float32),
            pltpu.VMEM((CHUNK * NG,), jnp.float32),
            pltpu.VMEM((2 * NG, 16), jnp.float32),
            pltpu.SemaphoreType.DMA,
            pltpu.SemaphoreType.DMA,
        ],
    )
    return fn(ei, posf, prm)


def kernel(pos, edge_index, shift, scale):
    coef = -1.0 / (2.0 * scale * scale)
    prm = jnp.concatenate(
        [jnp.broadcast_to(shift[:, None], (NG, 16)),
         jnp.broadcast_to(coef[:, None], (NG, 16))], axis=0)
    out = _gbf_sc(edge_index.reshape(-1), pos.reshape(-1), prm)
    return out.reshape(N_EDGES, NG)


# speedup vs baseline: 2.9091x; 2.9091x over previous
import jax
import jax.numpy as jnp
from jax import lax
from jax.experimental import pallas as pl
from jax.experimental.pallas import tpu as pltpu
from jax.experimental.pallas import tpu_sc as plsc

N_NODES = 100000
N_EDGES = 3200000
NG = 10

NUM_CORES = 2
NUM_SUBCORES = 16
N_WORKERS = NUM_CORES * NUM_SUBCORES
CHUNK = 1024
GROUPS = CHUNK // 16
TOTAL_CHUNKS = N_EDGES // CHUNK
BASE_CHUNKS = TOTAL_CHUNKS // N_WORKERS
EXTRA_CHUNKS = TOTAL_CHUNKS % N_WORKERS

_MAGIC = 0x5F3759DF


def _edge_body(ei_hbm, posf_hbm, prm_hbm, out_hbm,
               eiv, idxi, idxj, pif, pjf, outv, prmv, sem_a, sem_b):
    c = lax.axis_index("c")
    s = lax.axis_index("s")
    wid = s * NUM_CORES + c

    pltpu.sync_copy(prm_hbm, prmv)
    shifts = [prmv[k] for k in range(NG)]
    coefs = [prmv[NG + k] for k in range(NG)]
    n_chunks = BASE_CHUNKS + jnp.where(wid < EXTRA_CHUNKS, 1, 0)

    def chunk_body(ci, carry):
        chunk_id = ci * N_WORKERS + wid
        base = chunk_id * CHUNK
        pltpu.sync_copy(ei_hbm.at[:, pl.ds(base, CHUNK)], eiv)

        def expand(g, carry2):
            sl = pl.ds(g * 16, 16)
            s3 = eiv[0, sl] * 3
            d3 = eiv[1, sl] * 3
            idxi[sl] = s3
            idxi[pl.ds(CHUNK + g * 16, 16)] = s3 + 1
            idxi[pl.ds(2 * CHUNK + g * 16, 16)] = s3 + 2
            idxj[sl] = d3
            idxj[pl.ds(CHUNK + g * 16, 16)] = d3 + 1
            idxj[pl.ds(2 * CHUNK + g * 16, 16)] = d3 + 2
            return carry2

        lax.fori_loop(0, GROUPS, expand, 0)
        cp_i = pltpu.async_copy(posf_hbm.at[idxi], pif, sem_a)
        cp_j = pltpu.async_copy(posf_hbm.at[idxj], pjf, sem_b)
        cp_i.wait()
        cp_j.wait()

        def grp(g, carry2):
            sl = pl.ds(g * 16, 16)
            sly = pl.ds(CHUNK + g * 16, 16)
            slz = pl.ds(2 * CHUNK + g * 16, 16)
            dx = pif[sl] - pjf[sl]
            dy = pif[sly] - pjf[sly]
            dz = pif[slz] - pjf[slz]
            d2 = jnp.maximum(dx * dx + dy * dy + dz * dz,
                             jnp.float32(1e-30))
            bits = lax.bitcast_convert_type(d2, jnp.int32)
            r = lax.bitcast_convert_type(
                jnp.int32(_MAGIC) - lax.shift_right_logical(bits, 1),
                jnp.float32)
            half = jnp.float32(0.5) * d2
            for _ in range(3):
                r = r * (jnp.float32(1.5) - half * r * r)
            length = d2 * r
            for k in range(NG):
                t = length - shifts[k]
                o = jnp.exp(t * t * coefs[k])
                outv[k, sl] = o
            return carry2

        lax.fori_loop(0, GROUPS, grp, 0)
        pltpu.sync_copy(outv, out_hbm.at[:, pl.ds(base, CHUNK)])
        return carry

    lax.fori_loop(0, n_chunks, chunk_body, 0)


@jax.jit
def _gbf_sc(ei, posf, prm):
    mesh = plsc.VectorSubcoreMesh(core_axis_name="c", subcore_axis_name="s")
    fn = pl.kernel(
        _edge_body,
        out_type=jax.ShapeDtypeStruct((NG, N_EDGES), jnp.float32),
        mesh=mesh,
        compiler_params=pltpu.CompilerParams(needs_layout_passes=False),
        scratch_types=[
            pltpu.VMEM((2, CHUNK), jnp.int32),
            pltpu.VMEM((3 * CHUNK,), jnp.int32),
            pltpu.VMEM((3 * CHUNK,), jnp.int32),
            pltpu.VMEM((3 * CHUNK,), jnp.float32),
            pltpu.VMEM((3 * CHUNK,), jnp.float32),
            pltpu.VMEM((NG, CHUNK), jnp.float32),
            pltpu.VMEM((2 * NG, 16), jnp.float32),
            pltpu.SemaphoreType.DMA,
            pltpu.SemaphoreType.DMA,
        ],
    )
    return fn(ei, posf, prm)


def kernel(pos, edge_index, shift, scale):
    coef = -1.0 / (2.0 * scale * scale)
    prm = jnp.concatenate(
        [jnp.broadcast_to(shift[:, None], (NG, 16)),
         jnp.broadcast_to(coef[:, None], (NG, 16))], axis=0)
    out = _gbf_sc(edge_index, pos.reshape(-1), prm)
    return out.T


# speedup vs baseline: 3.7072x; 1.2743x over previous
import jax
import jax.numpy as jnp
from jax import lax
from jax.experimental import pallas as pl
from jax.experimental.pallas import tpu as pltpu
from jax.experimental.pallas import tpu_sc as plsc

N_NODES = 100000
N_EDGES = 3200000
NG = 10

NUM_CORES = 2
NUM_SUBCORES = 16
N_WORKERS = NUM_CORES * NUM_SUBCORES
CHUNK = 1024
GROUPS = CHUNK // 16
TOTAL_CHUNKS = N_EDGES // CHUNK
BASE_CHUNKS = TOTAL_CHUNKS // N_WORKERS
EXTRA_CHUNKS = TOTAL_CHUNKS % N_WORKERS
N_PAIRS = (BASE_CHUNKS + 1 + 1) // 2

_MAGIC = 0x5F3759DF


def _edge_body(ei_hbm, posf_hbm, prm_hbm, out_hbm,
               eiv0, eiv1, idxi0, idxi1, idxj0, idxj1,
               pif0, pif1, pjf0, pjf1, outv0, outv1, prmv,
               sem_e0, sem_e1, sem_i0, sem_i1, sem_j0, sem_j1,
               sem_o0, sem_o1):
    c = lax.axis_index("c")
    s = lax.axis_index("s")
    wid = s * NUM_CORES + c

    eiv = [eiv0, eiv1]
    idxi = [idxi0, idxi1]
    idxj = [idxj0, idxj1]
    pif = [pif0, pif1]
    pjf = [pjf0, pjf1]
    outv = [outv0, outv1]
    sem_e = [sem_e0, sem_e1]
    sem_i = [sem_i0, sem_i1]
    sem_j = [sem_j0, sem_j1]
    sem_o = [sem_o0, sem_o1]

    pltpu.sync_copy(prm_hbm, prmv)
    shifts = [prmv[k] for k in range(NG)]
    coefs = [prmv[NG + k] for k in range(NG)]
    n_chunks = BASE_CHUNKS + jnp.where(wid < EXTRA_CHUNKS, 1, 0)

    def ei_slice(ci):
        return ei_hbm.at[:, pl.ds((ci * N_WORKERS + wid) * CHUNK, CHUNK)]

    def out_slice(ci):
        return out_hbm.at[:, pl.ds((ci * N_WORKERS + wid) * CHUNK, CHUNK)]

    def expand(p):
        ev, ii, ij = eiv[p], idxi[p], idxj[p]

        def body(g, carry):
            sl = pl.ds(g * 16, 16)
            s3 = ev[0, sl] * 3
            d3 = ev[1, sl] * 3
            ii[sl] = s3
            ii[pl.ds(CHUNK + g * 16, 16)] = s3 + 1
            ii[pl.ds(2 * CHUNK + g * 16, 16)] = s3 + 2
            ij[sl] = d3
            ij[pl.ds(CHUNK + g * 16, 16)] = d3 + 1
            ij[pl.ds(2 * CHUNK + g * 16, 16)] = d3 + 2
            return carry

        lax.fori_loop(0, GROUPS, body, 0)

    def fire_gathers(p):
        pltpu.async_copy(posf_hbm.at[idxi[p]], pif[p], sem_i[p])
        pltpu.async_copy(posf_hbm.at[idxj[p]], pjf[p], sem_j[p])

    def wait_gathers(p):
        pltpu.make_async_copy(posf_hbm.at[idxi[p]], pif[p], sem_i[p]).wait()
        pltpu.make_async_copy(posf_hbm.at[idxj[p]], pjf[p], sem_j[p]).wait()

    def compute(p):
        src, dst, ov = pif[p], pjf[p], outv[p]

        def body(g, carry):
            sl = pl.ds(g * 16, 16)
            sly = pl.ds(CHUNK + g * 16, 16)
            slz = pl.ds(2 * CHUNK + g * 16, 16)
            dx = src[sl] - dst[sl]
            dy = src[sly] - dst[sly]
            dz = src[slz] - dst[slz]
            d2 = jnp.maximum(dx * dx + dy * dy + dz * dz,
                             jnp.float32(1e-30))
            bits = lax.bitcast_convert_type(d2, jnp.int32)
            r = lax.bitcast_convert_type(
                jnp.int32(_MAGIC) - lax.shift_right_logical(bits, 1),
                jnp.float32)
            half = jnp.float32(0.5) * d2
            for _ in range(3):
                r = r * (jnp.float32(1.5) - half * r * r)
            length = d2 * r
            for k in range(NG):
                t = length - shifts[k]
                o = jnp.exp(t * t * coefs[k])
                ov[k, sl] = o
            return carry

        lax.fori_loop(0, GROUPS, body, 0)

    def step(n, p):
        q = 1 - p
        wait_gathers(p)

        @pl.when(n + 1 < n_chunks)
        def _prep_next():
            pltpu.make_async_copy(ei_slice(n + 1), eiv[q], sem_e[q]).wait()
            expand(q)
            fire_gathers(q)

        @pl.when(n + 2 < n_chunks)
        def _prefetch_ei():
            pltpu.async_copy(ei_slice(n + 2), eiv[p], sem_e[p])

        @pl.when(n >= 2)
        def _drain_out():
            pltpu.make_async_copy(outv[p], out_slice(n - 2), sem_o[p]).wait()

        compute(p)
        pltpu.async_copy(outv[p], out_slice(n), sem_o[p])

    pltpu.sync_copy(ei_slice(0), eiv[0])
    expand(0)
    fire_gathers(0)
    pltpu.async_copy(ei_slice(1), eiv[1], sem_e[1])

    def pair_body(i, carry):
        step(2 * i, 0)

        @pl.when(2 * i + 1 < n_chunks)
        def _odd():
            step(2 * i + 1, 1)

        return carry

    lax.fori_loop(0, N_PAIRS, pair_body, 0)
    pltpu.make_async_copy(outv[0], out_slice(0), sem_o[0]).wait()
    pltpu.make_async_copy(outv[1], out_slice(0), sem_o[1]).wait()


@jax.jit
def _gbf_sc(ei, posf, prm):
    mesh = plsc.VectorSubcoreMesh(core_axis_name="c", subcore_axis_name="s")
    fn = pl.kernel(
        _edge_body,
        out_type=jax.ShapeDtypeStruct((NG, N_EDGES), jnp.float32),
        mesh=mesh,
        compiler_params=pltpu.CompilerParams(needs_layout_passes=False),
        scratch_types=[
            pltpu.VMEM((2, CHUNK), jnp.int32),
            pltpu.VMEM((2, CHUNK), jnp.int32),
            pltpu.VMEM((3 * CHUNK,), jnp.int32),
            pltpu.VMEM((3 * CHUNK,), jnp.int32),
            pltpu.VMEM((3 * CHUNK,), jnp.int32),
            pltpu.VMEM((3 * CHUNK,), jnp.int32),
            pltpu.VMEM((3 * CHUNK,), jnp.float32),
            pltpu.VMEM((3 * CHUNK,), jnp.float32),
            pltpu.VMEM((3 * CHUNK,), jnp.float32),
            pltpu.VMEM((3 * CHUNK,), jnp.float32),
            pltpu.VMEM((NG, CHUNK), jnp.float32),
            pltpu.VMEM((NG, CHUNK), jnp.float32),
            pltpu.VMEM((2 * NG, 16), jnp.float32),
            pltpu.SemaphoreType.DMA,
            pltpu.SemaphoreType.DMA,
            pltpu.SemaphoreType.DMA,
            pltpu.SemaphoreType.DMA,
            pltpu.SemaphoreType.DMA,
            pltpu.SemaphoreType.DMA,
            pltpu.SemaphoreType.DMA,
            pltpu.SemaphoreType.DMA,
        ],
    )
    return fn(ei, posf, prm)


def kernel(pos, edge_index, shift, scale):
    coef = -1.0 / (2.0 * scale * scale)
    prm = jnp.concatenate(
        [jnp.broadcast_to(shift[:, None], (NG, 16)),
         jnp.broadcast_to(coef[:, None], (NG, 16))], axis=0)
    out = _gbf_sc(edge_index, pos.reshape(-1), prm)
    return out.T
